# Initial kernel scaffold; baseline (speedup 1.0000x reference)
import jax
import jax.numpy as jnp
from jax.experimental import pallas as pl


def kernel(x, mask):
    raise NotImplementedError("write your pallas kernel here")


# speedup vs baseline: 3.8434x; 3.8434x over previous
import functools

import jax
import jax.numpy as jnp
from jax.experimental import pallas as pl


_ROWS = 65536
_COLS = 512
_BLK_ROWS = 1024
_CHUNK = 128


def _body(x_ref, m_ref, o_ref):
    xb = x_ref[...]
    mb = m_ref[...]
    mx = jnp.where(mb, xb, 0.0)
    r = jax.lax.broadcasted_iota(jnp.int32, (_CHUNK, _CHUNK), 0)
    c = jax.lax.broadcasted_iota(jnp.int32, (_CHUNK, _CHUNK), 1)
    tri = jnp.where(r <= c, 1.0, 0.0).astype(jnp.float32)
    carry = jnp.zeros((xb.shape[0], 1), dtype=jnp.float32)
    for j in range(_COLS // _CHUNK):
        seg = mx[:, j * _CHUNK:(j + 1) * _CHUNK]
        pj = jax.lax.dot_general(
            seg, tri, (((1,), (0,)), ((), ())),
            preferred_element_type=jnp.float32)
        o_ref[:, j * _CHUNK:(j + 1) * _CHUNK] = pj + carry
        carry = carry + jnp.sum(seg, axis=1, keepdims=True)


@jax.jit
def kernel(x, mask):
    grid = (_ROWS // _BLK_ROWS,)
    spec = pl.BlockSpec((_BLK_ROWS, _COLS), lambda i: (i, 0))
    return pl.pallas_call(
        _body,
        grid=grid,
        in_specs=[spec, spec],
        out_specs=spec,
        out_shape=jax.ShapeDtypeStruct((_ROWS, _COLS), jnp.float32),
    )(x, mask)
